# baseline (speedup 1.0000x reference)
import jax
import jax.numpy as jnp
from jax import lax
from jax.experimental import pallas as pl
from jax.experimental.pallas import tpu as pltpu
from jax.experimental.pallas import tpu_sc as plsc
from jax.experimental import layout as jex_layout

NUM_NODES = 200000
MAX_ELS = 200001
B = 128
NC = 2
NS = 16
NW = NC * NS
CHUNK_OUT = 64
CHUNK_IDX = 2 * CHUNK_OUT
NCHUNKS = NUM_NODES // CHUNK_OUT
NBUF = 3
NITER = -(-NCHUNKS // (NW * NBUF))
LANES = 16
DGROUPS = B // LANES
IDX_GROUPS = CHUNK_IDX // LANES


def _body(node_hbm, cids_hbm, em_hbm, out_hbm,
          c0, c1, c2, b0, b1, b2, o0, o1, o2, row_v,
          f0, f1, f2, g0, g1, g2, s0, s1, s2):
    bufs = (
        (c0, f0, b0, g0, o0, s0),
        (c1, f1, b1, g1, o1, s1),
        (c2, f2, b2, g2, o2, s2),
    )
    c = lax.axis_index("c")
    s = lax.axis_index("s")
    wid = s * NC + c
    def fetch_idx(t, ci2, fsem):
        cid = wid + t * NW
        pltpu.async_copy(cids_hbm.at[cid], ci2, fsem)

    def wait_idx(ci2, fsem):
        pltpu.make_async_copy(cids_hbm.at[0], ci2, fsem).wait()

    def start_gather(ci2, buf, gsem):
        pltpu.async_copy(node_hbm.at[ci2], buf, gsem)

    for p in range(NBUF):
        fetch_idx(p, bufs[p][0], bufs[p][1])
    for p in range(NBUF):
        wait_idx(bufs[p][0], bufs[p][1])
        start_gather(bufs[p][0], bufs[p][2], bufs[p][3])

    def step(i, carry):
        for p in range(NBUF):
            ci2, fsem, buf, gsem, out_v, osem = bufs[p]
            t = i * NBUF + p
            cid = wid + t * NW
            valid = cid < NCHUNKS
            valid_next = cid + NBUF * NW < NCHUNKS

            @pl.when(valid)
            def _():
                pltpu.make_async_copy(node_hbm.at[ci2], buf, gsem).wait()

            @pl.when(valid_next)
            def _():
                fetch_idx(t + NBUF, ci2, fsem)

            @pl.when(valid)
            def _():
                @pl.when(i >= 1)
                def _():
                    pltpu.make_async_copy(
                        out_v, out_hbm.at[pl.ds(0, CHUNK_OUT)], osem
                    ).wait()

                @plsc.parallel_loop(0, CHUNK_OUT, step=1, unroll=4)
                def rowfn(j):
                    for d in range(DGROUPS):
                        sl = pl.ds(d * LANES, LANES)
                        out_v[j, sl] = buf[2 * j, sl] + buf[2 * j + 1, sl]

                pltpu.async_copy(
                    out_v, out_hbm.at[pl.ds(cid * CHUNK_OUT, CHUNK_OUT)], osem
                )

            @pl.when(valid_next)
            def _():
                wait_idx(ci2, fsem)
                start_gather(ci2, buf, gsem)
        return carry

    lax.fori_loop(0, NITER, step, 0)

    for p in range(NBUF):
        t = (NITER - 1) * NBUF + p

        @pl.when(wid + t * NW < NCHUNKS)
        def _(p=p):
            pltpu.make_async_copy(
                bufs[p][4], out_hbm.at[pl.ds(0, CHUNK_OUT)], bufs[p][5]
            ).wait()

    @pl.when(wid == 0)
    def _():
        pltpu.sync_copy(em_hbm.at[pl.ds(NUM_NODES, 1)], row_v)
        pltpu.sync_copy(row_v, out_hbm.at[pl.ds(NUM_NODES, 1)])


@jax.jit
def _run(node_mars, element_mars, cids):
    mesh = plsc.VectorSubcoreMesh(
        core_axis_name="c", subcore_axis_name="s", num_cores=NC, num_subcores=NS
    )
    return pl.kernel(
        _body,
        out_type=jax.ShapeDtypeStruct((MAX_ELS, B), jnp.float32),
        mesh=mesh,
        scratch_types=[
            pltpu.VMEM((CHUNK_IDX,), jnp.int32),
            pltpu.VMEM((CHUNK_IDX,), jnp.int32),
            pltpu.VMEM((CHUNK_IDX,), jnp.int32),
            pltpu.VMEM((CHUNK_IDX, B), jnp.float32),
            pltpu.VMEM((CHUNK_IDX, B), jnp.float32),
            pltpu.VMEM((CHUNK_IDX, B), jnp.float32),
            pltpu.VMEM((CHUNK_OUT, B), jnp.float32),
            pltpu.VMEM((CHUNK_OUT, B), jnp.float32),
            pltpu.VMEM((CHUNK_OUT, B), jnp.float32),
            pltpu.VMEM((1, B), jnp.float32),
            pltpu.SemaphoreType.DMA,
            pltpu.SemaphoreType.DMA,
            pltpu.SemaphoreType.DMA,
            pltpu.SemaphoreType.DMA,
            pltpu.SemaphoreType.DMA,
            pltpu.SemaphoreType.DMA,
            pltpu.SemaphoreType.DMA,
            pltpu.SemaphoreType.DMA,
            pltpu.SemaphoreType.DMA,
        ],
    )(node_mars, cids, element_mars)


def kernel(node_mars, element_mars, scratch, nids, cids):
    cids_dense = jex_layout.with_layout_constraint(
        cids, jex_layout.Layout((0, 1), tiling=((1, 2),))
    )
    return _run(node_mars, element_mars, cids_dense.reshape(NCHUNKS, CHUNK_IDX))


# speedup vs baseline: 1.0492x; 1.0492x over previous
import jax
import jax.numpy as jnp
from jax import lax
from jax.experimental import pallas as pl
from jax.experimental.pallas import tpu as pltpu
from jax.experimental.pallas import tpu_sc as plsc

NUM_NODES = 200000
MAX_ELS = 200001
B = 128
NC = 2
NS = 16
NW = NC * NS
CHUNK_OUT = 64
CHUNK_IDX = 2 * CHUNK_OUT
NCHUNKS = NUM_NODES // CHUNK_OUT
NBUF = 3
LANES = 16
DGROUPS = B // LANES
NSPLIT = 2
SPLIT_SIZES = [NCHUNKS // NSPLIT + (1 if k < NCHUNKS % NSPLIT else 0)
               for k in range(NSPLIT)]
SPLIT_BASES = [sum(SPLIT_SIZES[:k]) for k in range(NSPLIT)]

_SCRATCH = [
    pltpu.VMEM((CHUNK_IDX,), jnp.int32),
    pltpu.VMEM((CHUNK_IDX,), jnp.int32),
    pltpu.VMEM((CHUNK_IDX,), jnp.int32),
    pltpu.VMEM((CHUNK_IDX, B), jnp.float32),
    pltpu.VMEM((CHUNK_IDX, B), jnp.float32),
    pltpu.VMEM((CHUNK_IDX, B), jnp.float32),
    pltpu.VMEM((CHUNK_OUT, B), jnp.float32),
    pltpu.VMEM((CHUNK_OUT, B), jnp.float32),
    pltpu.VMEM((CHUNK_OUT, B), jnp.float32),
    pltpu.VMEM((1, B), jnp.float32),
    pltpu.SemaphoreType.DMA,
    pltpu.SemaphoreType.DMA,
    pltpu.SemaphoreType.DMA,
    pltpu.SemaphoreType.DMA,
    pltpu.SemaphoreType.DMA,
    pltpu.SemaphoreType.DMA,
    pltpu.SemaphoreType.DMA,
    pltpu.SemaphoreType.DMA,
    pltpu.SemaphoreType.DMA,
]


def _make_body(chunk_base, nch, last_slice):
    niter = -(-nch // (NW * NBUF))

    def _body(node_hbm, idx_hbm, em_hbm, out_hbm,
              c0, c1, c2, b0, b1, b2, o0, o1, o2, row_v,
              f0, f1, f2, g0, g1, g2, s0, s1, s2):
        bufs = (
            (c0, f0, b0, g0, o0, s0),
            (c1, f1, b1, g1, o1, s1),
            (c2, f2, b2, g2, o2, s2),
        )
        c = lax.axis_index("c")
        s = lax.axis_index("s")
        wid = s * NC + c

        def fetch_idx(t, ci2, fsem):
            pltpu.async_copy(idx_hbm.at[wid + t * NW], ci2, fsem)

        def wait_idx(ci2, fsem):
            pltpu.make_async_copy(idx_hbm.at[0], ci2, fsem).wait()

        def start_gather(ci2, buf, gsem):
            pltpu.async_copy(node_hbm.at[ci2], buf, gsem)

        for p in range(NBUF):
            fetch_idx(p, bufs[p][0], bufs[p][1])
        for p in range(NBUF):
            wait_idx(bufs[p][0], bufs[p][1])
            start_gather(bufs[p][0], bufs[p][2], bufs[p][3])

        def step(i, carry):
            for p in range(NBUF):
                ci2, fsem, buf, gsem, out_v, osem = bufs[p]
                t = i * NBUF + p
                lid = wid + t * NW
                valid = lid < nch
                valid_next = lid + NBUF * NW < nch

                @pl.when(valid)
                def _():
                    pltpu.make_async_copy(node_hbm.at[ci2], buf, gsem).wait()

                @pl.when(valid_next)
                def _():
                    fetch_idx(t + NBUF, ci2, fsem)

                @pl.when(valid)
                def _():
                    @pl.when(i >= 1)
                    def _():
                        pltpu.make_async_copy(
                            out_v, out_hbm.at[pl.ds(0, CHUNK_OUT)], osem
                        ).wait()

                    @plsc.parallel_loop(0, CHUNK_OUT, step=1, unroll=4)
                    def rowfn(j):
                        for d in range(DGROUPS):
                            sl = pl.ds(d * LANES, LANES)
                            out_v[j, sl] = buf[2 * j, sl] + buf[2 * j + 1, sl]

                    base = (chunk_base + lid) * CHUNK_OUT
                    pltpu.async_copy(
                        out_v, out_hbm.at[pl.ds(base, CHUNK_OUT)], osem
                    )

                @pl.when(valid_next)
                def _():
                    wait_idx(ci2, fsem)
                    start_gather(ci2, buf, gsem)
            return carry

        lax.fori_loop(0, niter, step, 0)

        for p in range(NBUF):
            t = (niter - 1) * NBUF + p

            @pl.when(wid + t * NW < nch)
            def _(p=p):
                pltpu.make_async_copy(
                    bufs[p][4], out_hbm.at[pl.ds(0, CHUNK_OUT)], bufs[p][5]
                ).wait()

        if last_slice:
            @pl.when(wid == 0)
            def _():
                pltpu.sync_copy(em_hbm.at[pl.ds(NUM_NODES, 1)], row_v)
                pltpu.sync_copy(row_v, out_hbm.at[pl.ds(NUM_NODES, 1)])

    return _body


_MESH = plsc.VectorSubcoreMesh(
    core_axis_name="c", subcore_axis_name="s", num_cores=NC, num_subcores=NS
)

_SLICE_KERNELS = [
    pl.kernel(
        _make_body(SPLIT_BASES[k], SPLIT_SIZES[k], k == NSPLIT - 1),
        out_type=(),
        mesh=_MESH,
        scratch_types=_SCRATCH,
    )
    for k in range(NSPLIT)
]


@jax.jit
def _run(node_mars, element_mars, cids):
    out_ref = jax.empty_ref(jax.ShapeDtypeStruct((MAX_ELS, B), jnp.float32))
    for k in range(NSPLIT):
        lo = SPLIT_BASES[k] * CHUNK_OUT
        hi = (SPLIT_BASES[k] + SPLIT_SIZES[k]) * CHUNK_OUT
        idx_k = cids[lo:hi].reshape(SPLIT_SIZES[k], CHUNK_IDX)
        _SLICE_KERNELS[k](node_mars, idx_k, element_mars, out_ref)
    return out_ref[...]


def kernel(node_mars, element_mars, scratch, nids, cids):
    return _run(node_mars, element_mars, cids)


# speedup vs baseline: 1.7057x; 1.6258x over previous
import jax
import jax.numpy as jnp
from jax import lax
from jax.experimental import pallas as pl
from jax.experimental.pallas import tpu as pltpu
from jax.experimental.pallas import tpu_sc as plsc

NUM_NODES = 200000
MAX_ELS = 200001
B = 128
NC = 2
NS = 16
NW = NC * NS
SC_OUT = 128
NSCH = NUM_NODES // SC_OUT
TAIL = NUM_NODES - NSCH * SC_OUT
NBUF = 2
NITER = -(-NSCH // (NW * NBUF))
LANES = 16
DGROUPS = B // LANES


def _body(node_hbm, cidsT_hbm, em_hbm, out_hbm,
          ia0, ib0, ia1, ib1, a0, b0, a1, b1, o0, o1, row_v,
          f0, f1, g0, g1, s0, s1):
    bufs = (
        (ia0, ib0, f0, a0, b0, g0, o0, s0),
        (ia1, ib1, f1, a1, b1, g1, o1, s1),
    )
    c = lax.axis_index("c")
    s = lax.axis_index("s")
    wid = s * NC + c

    def fetch_idx(t, ia, ib, fsem):
        off = (wid + t * NW) * SC_OUT
        pltpu.async_copy(cidsT_hbm.at[0, pl.ds(off, SC_OUT)], ia, fsem)
        pltpu.async_copy(cidsT_hbm.at[1, pl.ds(off, SC_OUT)], ib, fsem)

    def wait_idx(ia, ib, fsem):
        pltpu.make_async_copy(cidsT_hbm.at[0, pl.ds(0, SC_OUT)], ia, fsem).wait()
        pltpu.make_async_copy(cidsT_hbm.at[1, pl.ds(0, SC_OUT)], ib, fsem).wait()

    def start_gather(ia, ib, bufa, bufb, gsem):
        pltpu.async_copy(node_hbm.at[ia], bufa, gsem)
        pltpu.async_copy(node_hbm.at[ib], bufb, gsem)

    def wait_gather(ia, ib, bufa, bufb, gsem):
        pltpu.make_async_copy(node_hbm.at[ia], bufa, gsem).wait()
        pltpu.make_async_copy(node_hbm.at[ib], bufb, gsem).wait()

    for p in range(NBUF):
        fetch_idx(p, bufs[p][0], bufs[p][1], bufs[p][2])
    for p in range(NBUF):
        wait_idx(bufs[p][0], bufs[p][1], bufs[p][2])
        start_gather(bufs[p][0], bufs[p][1], bufs[p][3], bufs[p][4], bufs[p][5])

    def step(i, carry):
        for p in range(NBUF):
            ia, ib, fsem, bufa, bufb, gsem, out_v, osem = bufs[p]
            t = i * NBUF + p
            sid = wid + t * NW
            valid = sid < NSCH
            valid_next = sid + NBUF * NW < NSCH

            @pl.when(valid)
            def _():
                wait_gather(ia, ib, bufa, bufb, gsem)

            @pl.when(valid_next)
            def _():
                fetch_idx(t + NBUF, ia, ib, fsem)

            @pl.when(valid)
            def _():
                @pl.when(i >= 1)
                def _():
                    pltpu.make_async_copy(
                        out_v, out_hbm.at[pl.ds(0, SC_OUT)], osem
                    ).wait()

                @plsc.parallel_loop(0, SC_OUT, step=1, unroll=4)
                def rowfn(j):
                    for d in range(DGROUPS):
                        sl = pl.ds(d * LANES, LANES)
                        out_v[j, sl] = bufa[j, sl] + bufb[j, sl]

                pltpu.async_copy(
                    out_v, out_hbm.at[pl.ds(sid * SC_OUT, SC_OUT)], osem
                )

            @pl.when(valid_next)
            def _():
                wait_idx(ia, ib, fsem)
                start_gather(ia, ib, bufa, bufb, gsem)
        return carry

    lax.fori_loop(0, NITER, step, 0)

    for p in range(NBUF):
        t = (NITER - 1) * NBUF + p

        @pl.when(wid + t * NW < NSCH)
        def _(p=p):
            pltpu.make_async_copy(
                bufs[p][6], out_hbm.at[pl.ds(0, SC_OUT)], bufs[p][7]
            ).wait()

    @pl.when(wid == 0)
    def _():
        ia, ib, fsem, bufa, bufb, gsem, out_v, osem = bufs[0]
        toff = NSCH * SC_OUT
        pltpu.async_copy(cidsT_hbm.at[0, pl.ds(toff, SC_OUT)], ia, fsem)
        pltpu.async_copy(cidsT_hbm.at[1, pl.ds(toff, SC_OUT)], ib, fsem)
        wait_idx(ia, ib, fsem)
        start_gather(ia, ib, bufa, bufb, gsem)
        wait_gather(ia, ib, bufa, bufb, gsem)

        @plsc.parallel_loop(0, TAIL, step=1, unroll=4)
        def tailfn(j):
            for d in range(DGROUPS):
                sl = pl.ds(d * LANES, LANES)
                out_v[j, sl] = bufa[j, sl] + bufb[j, sl]

        pltpu.sync_copy(
            out_v.at[pl.ds(0, TAIL)], out_hbm.at[pl.ds(toff, TAIL)]
        )
        pltpu.sync_copy(em_hbm.at[pl.ds(NUM_NODES, 1)], row_v)
        pltpu.sync_copy(row_v, out_hbm.at[pl.ds(NUM_NODES, 1)])


@jax.jit
def _run(node_mars, element_mars, cidsT):
    mesh = plsc.VectorSubcoreMesh(
        core_axis_name="c", subcore_axis_name="s", num_cores=NC, num_subcores=NS
    )
    return pl.kernel(
        _body,
        out_type=jax.ShapeDtypeStruct((MAX_ELS, B), jnp.float32),
        mesh=mesh,
        scratch_types=[
            pltpu.VMEM((SC_OUT,), jnp.int32),
            pltpu.VMEM((SC_OUT,), jnp.int32),
            pltpu.VMEM((SC_OUT,), jnp.int32),
            pltpu.VMEM((SC_OUT,), jnp.int32),
            pltpu.VMEM((SC_OUT, B), jnp.float32),
            pltpu.VMEM((SC_OUT, B), jnp.float32),
            pltpu.VMEM((SC_OUT, B), jnp.float32),
            pltpu.VMEM((SC_OUT, B), jnp.float32),
            pltpu.VMEM((SC_OUT, B), jnp.float32),
            pltpu.VMEM((SC_OUT, B), jnp.float32),
            pltpu.VMEM((1, B), jnp.float32),
            pltpu.SemaphoreType.DMA,
            pltpu.SemaphoreType.DMA,
            pltpu.SemaphoreType.DMA,
            pltpu.SemaphoreType.DMA,
            pltpu.SemaphoreType.DMA,
            pltpu.SemaphoreType.DMA,
        ],
    )(node_mars, cidsT, element_mars)


def kernel(node_mars, element_mars, scratch, nids, cids):
    cidsT = jnp.pad(cids.T, ((0, 0), (0, SC_OUT - TAIL)))
    return _run(node_mars, element_mars, cidsT)
